# Initial kernel scaffold; baseline (speedup 1.0000x reference)
import jax
import jax.numpy as jnp
from jax.experimental import pallas as pl


def kernel(z_e_x, W):
    raise NotImplementedError("write your pallas kernel here")


# speedup vs baseline: 3.6021x; 3.6021x over previous
import jax
import jax.numpy as jnp
from jax.experimental import pallas as pl

N = 8192
K = 8192
D = 32
BN = 256


def _vq_kernel(z_ref, w_ref, logits_ref, idx_ref):
    logits = jax.lax.dot_general(
        z_ref[...], w_ref[...],
        dimension_numbers=(((1,), (1,)), ((), ())),
        preferred_element_type=jnp.float32,
    )
    logits_ref[...] = logits
    m = jnp.max(logits, axis=1, keepdims=True)
    e = jnp.exp(logits - m)
    y = e / jnp.sum(e, axis=1, keepdims=True)
    idx_ref[0, 0, :] = jnp.argmax(y, axis=1).astype(jnp.int32)


def kernel(z_e_x, W):
    grid = (N // BN,)
    logits, idx = pl.pallas_call(
        _vq_kernel,
        grid=grid,
        in_specs=[
            pl.BlockSpec((BN, D), lambda i: (i, 0)),
            pl.BlockSpec((K, D), lambda i: (0, 0)),
        ],
        out_specs=[
            pl.BlockSpec((BN, K), lambda i: (i, 0)),
            pl.BlockSpec((1, 1, BN), lambda i: (i, 0, 0)),
        ],
        out_shape=[
            jax.ShapeDtypeStruct((N, K), jnp.float32),
            jax.ShapeDtypeStruct((N // BN, 1, BN), jnp.int32),
        ],
    )(z_e_x, W)
    return (logits, idx.reshape(N))


# speedup vs baseline: 4.1511x; 1.1524x over previous
import jax
import jax.numpy as jnp
from jax.experimental import pallas as pl

N = 8192
K = 8192
D = 32
BN = 256


def _vq_kernel(z_ref, w_ref, logits_ref, idx_ref):
    logits = jax.lax.dot_general(
        z_ref[...], w_ref[...],
        dimension_numbers=(((1,), (1,)), ((), ())),
        preferred_element_type=jnp.float32,
    )
    logits_ref[...] = logits
    m = jnp.max(logits, axis=1, keepdims=True)
    e = jnp.exp(logits - m)
    s = jnp.sum(e, axis=1, keepdims=True)
    emax = jnp.exp(m - m)
    ymax = emax / s
    iota = jax.lax.broadcasted_iota(jnp.int32, e.shape, 1).astype(jnp.float32)
    cand = jnp.where(e / s == ymax, iota, jnp.float32(K))
    idx_ref[0, 0, :] = jnp.min(cand, axis=1).astype(jnp.int32)


def kernel(z_e_x, W):
    grid = (N // BN,)
    logits, idx = pl.pallas_call(
        _vq_kernel,
        grid=grid,
        in_specs=[
            pl.BlockSpec((BN, D), lambda i: (i, 0)),
            pl.BlockSpec((K, D), lambda i: (0, 0)),
        ],
        out_specs=[
            pl.BlockSpec((BN, K), lambda i: (i, 0)),
            pl.BlockSpec((1, 1, BN), lambda i: (i, 0, 0)),
        ],
        out_shape=[
            jax.ShapeDtypeStruct((N, K), jnp.float32),
            jax.ShapeDtypeStruct((N // BN, 1, BN), jnp.int32),
        ],
    )(z_e_x, W)
    return (logits, idx.reshape(N))


# speedup vs baseline: 4.1556x; 1.0011x over previous
import jax
import jax.numpy as jnp
from jax.experimental import pallas as pl
from jax.experimental.pallas import tpu as pltpu

N = 8192
K = 8192
D = 32
BN = 256


def _vq_kernel(z_ref, w_ref, logits_ref, idx_ref):
    logits = jax.lax.dot_general(
        z_ref[...], w_ref[...],
        dimension_numbers=(((1,), (1,)), ((), ())),
        preferred_element_type=jnp.float32,
    )
    logits_ref[...] = logits
    m = jnp.max(logits, axis=1, keepdims=True)
    e = jnp.exp(logits - m)
    s = jnp.sum(e, axis=1, keepdims=True)
    emax = jnp.exp(m - m)
    ymax = emax / s
    iota = jax.lax.broadcasted_iota(jnp.int32, e.shape, 1).astype(jnp.float32)
    cand = jnp.where(e / s == ymax, iota, jnp.float32(K))
    idx_ref[0, 0, :] = jnp.min(cand, axis=1).astype(jnp.int32)


def kernel(z_e_x, W):
    grid = (N // BN,)
    logits, idx = pl.pallas_call(
        _vq_kernel,
        grid=grid,
        in_specs=[
            pl.BlockSpec((BN, D), lambda i: (i, 0)),
            pl.BlockSpec((K, D), lambda i: (0, 0)),
        ],
        out_specs=[
            pl.BlockSpec((BN, K), lambda i: (i, 0)),
            pl.BlockSpec((1, 1, BN), lambda i: (i, 0, 0)),
        ],
        out_shape=[
            jax.ShapeDtypeStruct((N, K), jnp.float32),
            jax.ShapeDtypeStruct((N // BN, 1, BN), jnp.int32),
        ],
        compiler_params=pltpu.CompilerParams(
            dimension_semantics=("parallel",),
        ),
    )(z_e_x, W)
    return (logits, idx.reshape(N))


# speedup vs baseline: 4.4031x; 1.0596x over previous
import jax
import jax.numpy as jnp
from jax.experimental import pallas as pl
from jax.experimental.pallas import tpu as pltpu

N = 8192
K = 8192
D = 32
BN = 512


def _vq_kernel(z_ref, w_ref, logits_ref, idx_ref):
    logits = jax.lax.dot_general(
        z_ref[...], w_ref[...],
        dimension_numbers=(((1,), (1,)), ((), ())),
        preferred_element_type=jnp.float32,
    )
    logits_ref[...] = logits
    m = jnp.max(logits, axis=1, keepdims=True)
    e = jnp.exp(logits - m)
    s = jnp.sum(e, axis=1, keepdims=True)
    emax = jnp.exp(m - m)
    ymax = emax / s
    iota = jax.lax.broadcasted_iota(jnp.int32, e.shape, 1).astype(jnp.float32)
    cand = jnp.where(e / s == ymax, iota, jnp.float32(K))
    idx_ref[0, 0, :] = jnp.min(cand, axis=1).astype(jnp.int32)


def kernel(z_e_x, W):
    grid = (N // BN,)
    logits, idx = pl.pallas_call(
        _vq_kernel,
        grid=grid,
        in_specs=[
            pl.BlockSpec((BN, D), lambda i: (i, 0)),
            pl.BlockSpec((K, D), lambda i: (0, 0)),
        ],
        out_specs=[
            pl.BlockSpec((BN, K), lambda i: (i, 0)),
            pl.BlockSpec((1, 1, BN), lambda i: (i, 0, 0)),
        ],
        out_shape=[
            jax.ShapeDtypeStruct((N, K), jnp.float32),
            jax.ShapeDtypeStruct((N // BN, 1, BN), jnp.int32),
        ],
        compiler_params=pltpu.CompilerParams(
            dimension_semantics=("parallel",),
        ),
    )(z_e_x, W)
    return (logits, idx.reshape(N))
